# baseline (speedup 1.0000x reference)
import functools

import jax
import jax.numpy as jnp
from jax.experimental import pallas as pl
from jax.experimental.pallas import tpu as pltpu

R = 128
V = 100000
CHUNK = 8192
NC = 13

NEG_INF = float("-inf")
BIG_I32 = 2**31 - 1


@functools.cache
def _gumbel_tiled():
    noise_key = jax.random.key(42)
    noise = jax.random.exponential(noise_key, (R, V), dtype=jnp.float32)
    noise = jnp.clip(noise, 1e-10, None)
    g = -jnp.log(noise)
    g = jnp.pad(g, ((0, 0), (0, NC * CHUNK - V)), constant_values=-1e30)
    return jnp.stack(jnp.split(g, NC, axis=1), axis=0)


def _sample_kernel(logits_ref, g_ref, t_ref, val_ref, idx_ref,
                   acc_val, acc_chunk):
    j = pl.program_id(0)

    @pl.when(j == 0)
    def _init():
        acc_val[...] = jnp.full((R, CHUNK), NEG_INF, jnp.float32)
        acc_chunk[...] = jnp.zeros((R, CHUNK), jnp.int32)

    inv_t = 1.0 / t_ref[...]
    y = logits_ref[...] * inv_t + g_ref[0]

    def _update(yv):
        take = yv > acc_val[...]
        acc_chunk[...] = jnp.where(take, j, acc_chunk[...])
        acc_val[...] = jnp.maximum(acc_val[...], yv)

    @pl.when(j < NC - 1)
    def _main():
        _update(y)

    @pl.when(j == NC - 1)
    def _tail():
        lane = jax.lax.broadcasted_iota(jnp.int32, (R, CHUNK), 1)
        _update(jnp.where(lane < V - (NC - 1) * CHUNK, y, NEG_INF))

    @pl.when(j == NC - 1)
    def _finalize():
        vals = acc_val[...]
        row_max = jnp.max(vals, axis=1, keepdims=True)
        lane = jax.lax.broadcasted_iota(jnp.int32, (R, CHUNK), 1)
        cols = acc_chunk[...] * CHUNK + lane
        cand = jnp.where(vals == row_max, cols, BIG_I32)
        val_ref[...] = row_max
        idx_ref[...] = jnp.min(cand, axis=1, keepdims=True)


def kernel(logits, temperatures):
    g = _gumbel_tiled()
    t2 = temperatures.reshape(R, 1)
    _, idxs = pl.pallas_call(
        _sample_kernel,
        grid=(NC,),
        in_specs=[
            pl.BlockSpec((R, CHUNK), lambda j: (0, j)),
            pl.BlockSpec((1, R, CHUNK), lambda j: (j, 0, 0)),
            pl.BlockSpec((R, 1), lambda j: (0, 0)),
        ],
        out_specs=[
            pl.BlockSpec((R, 1), lambda j: (0, 0)),
            pl.BlockSpec((R, 1), lambda j: (0, 0)),
        ],
        out_shape=[
            jax.ShapeDtypeStruct((R, 1), jnp.float32),
            jax.ShapeDtypeStruct((R, 1), jnp.int32),
        ],
        scratch_shapes=[
            pltpu.VMEM((R, CHUNK), jnp.float32),
            pltpu.VMEM((R, CHUNK), jnp.int32),
        ],
    )(logits, g, t2)
    return idxs.reshape(R)


# speedup vs baseline: 1.2933x; 1.2933x over previous
import jax
import jax.numpy as jnp
from jax.experimental import pallas as pl
from jax.experimental.pallas import tpu as pltpu

R = 128
V = 100000
CHUNK = 4096
NC = 25
TILE_R = 16
TILE_L = 512
NT = (R // TILE_R) * (CHUNK // TILE_L)

KEY1 = 42

NEG_INF = float("-inf")
BIG_I32 = 2**31 - 1
INV_LN2 = 1.4426950408889634


def _threefry_bits_key042_x2(a1i, b1i):
    u32 = jnp.uint32
    ks = (u32(0), u32(KEY1), u32(KEY1 ^ 0x1BD11BDA))
    rot1 = (13, 15, 26, 6)
    rot2 = (17, 29, 16, 24)

    a0, b0 = a1i, b1i
    a1 = ((a1i << u32(13)) | (a1i >> u32(19))) ^ a0
    b1 = ((b1i << u32(13)) | (b1i >> u32(19))) ^ b0
    for r in rot1[1:]:
        a0 = a0 + a1
        b0 = b0 + b1
        a1 = (a1 << u32(r)) | (a1 >> u32(32 - r))
        b1 = (b1 << u32(r)) | (b1 >> u32(32 - r))
        a1 = a1 ^ a0
        b1 = b1 ^ b0
    a0 = a0 + ks[1]
    b0 = b0 + ks[1]
    a1 = a1 + ks[2] + u32(1)
    b1 = b1 + ks[2] + u32(1)
    for i in range(1, 5):
        for r in (rot1 if i % 2 == 0 else rot2):
            a0 = a0 + a1
            b0 = b0 + b1
            a1 = (a1 << u32(r)) | (a1 >> u32(32 - r))
            b1 = (b1 << u32(r)) | (b1 >> u32(32 - r))
            a1 = a1 ^ a0
            b1 = b1 ^ b0
        a0 = a0 + ks[(i + 1) % 3]
        b0 = b0 + ks[(i + 1) % 3]
        a1 = a1 + ks[(i + 2) % 3] + u32(i + 1)
        b1 = b1 + ks[(i + 2) % 3] + u32(i + 1)
    return a0 ^ a1, b0 ^ b1


def _sample_kernel(logits_ref, t_ref, val_ref, idx_ref,
                   acc_val, acc_chunk, s_ref):
    j = pl.program_id(0)

    @pl.when(j == 0)
    def _init():
        acc_val[...] = jnp.full((R, CHUNK), NEG_INF, jnp.float32)
        acc_chunk[...] = jnp.zeros((R, CHUNK), jnp.int32)
        s_ref[...] = jnp.float32(INV_LN2) / t_ref[...]

    row_iota = jax.lax.broadcasted_iota(jnp.uint32, (TILE_R, TILE_L), 0)
    lane_iota = jax.lax.broadcasted_iota(jnp.uint32, (TILE_R, TILE_L), 1)
    base = row_iota * jnp.uint32(V) + lane_iota
    lane_i32 = lane_iota.astype(jnp.int32)
    chunk_col0 = j * CHUNK

    def tile_at(t):
        r0 = (t >> 2) * TILE_R
        c0 = (t & 3) * (2 * TILE_L)
        return r0, c0

    def compute_y2(r0, c0):
        x1i_a = base + (r0 * V + chunk_col0 + c0 + KEY1).astype(jnp.uint32)
        x1i_b = x1i_a + jnp.uint32(TILE_L)
        bits_a, bits_b = _threefry_bits_key042_x2(x1i_a, x1i_b)
        s_tile = s_ref[pl.ds(r0, TILE_R), :]

        def finish(bits, c):
            xs = logits_ref[pl.ds(r0, TILE_R), pl.ds(c, TILE_L)]
            f = jax.lax.bitcast_convert_type(
                (bits >> jnp.uint32(9)) | jnp.uint32(0x3F800000), jnp.float32)
            n2 = jnp.maximum(-jnp.log2(2.0 - f), 1e-10)
            return xs * s_tile - jnp.log2(n2)

        return finish(bits_a, c0), finish(bits_b, c0 + TILE_L)

    def update(r0, c0, y):
        av = acc_val[pl.ds(r0, TILE_R), pl.ds(c0, TILE_L)]
        take = y > av
        ac = acc_chunk[pl.ds(r0, TILE_R), pl.ds(c0, TILE_L)]
        acc_chunk[pl.ds(r0, TILE_R), pl.ds(c0, TILE_L)] = jnp.where(take, j, ac)
        acc_val[pl.ds(r0, TILE_R), pl.ds(c0, TILE_L)] = jnp.maximum(av, y)

    @pl.when(j < NC - 1)
    def _main():
        def body(t, carry):
            r0, c0 = tile_at(t)
            ya, yb = compute_y2(r0, c0)
            update(r0, c0, ya)
            update(r0, c0 + TILE_L, yb)
            return carry

        jax.lax.fori_loop(0, NT // 2, body, 0)

    @pl.when(j == NC - 1)
    def _tail():
        def body(t, carry):
            r0, c0 = tile_at(t)
            ya, yb = compute_y2(r0, c0)
            ya = jnp.where(chunk_col0 + c0 + lane_i32 < V, ya, NEG_INF)
            yb = jnp.where(chunk_col0 + c0 + TILE_L + lane_i32 < V, yb, NEG_INF)
            update(r0, c0, ya)
            update(r0, c0 + TILE_L, yb)
            return carry

        jax.lax.fori_loop(0, NT // 2, body, 0)

    @pl.when(j == NC - 1)
    def _finalize():
        vals = acc_val[...]
        row_max = jnp.max(vals, axis=1, keepdims=True)
        full_lane = jax.lax.broadcasted_iota(jnp.int32, (R, CHUNK), 1)
        cols = acc_chunk[...] * CHUNK + full_lane
        cand = jnp.where(vals == row_max, cols, BIG_I32)
        val_ref[...] = row_max
        idx_ref[...] = jnp.min(cand, axis=1, keepdims=True)


def kernel(logits, temperatures):
    t2 = temperatures.reshape(R, 1)
    _, idxs = pl.pallas_call(
        _sample_kernel,
        grid=(NC,),
        in_specs=[
            pl.BlockSpec((R, CHUNK), lambda j: (0, j)),
            pl.BlockSpec((R, 1), lambda j: (0, 0)),
        ],
        out_specs=[
            pl.BlockSpec((R, 1), lambda j: (0, 0)),
            pl.BlockSpec((R, 1), lambda j: (0, 0)),
        ],
        out_shape=[
            jax.ShapeDtypeStruct((R, 1), jnp.float32),
            jax.ShapeDtypeStruct((R, 1), jnp.int32),
        ],
        scratch_shapes=[
            pltpu.VMEM((R, CHUNK), jnp.float32),
            pltpu.VMEM((R, CHUNK), jnp.int32),
            pltpu.VMEM((R, 1), jnp.float32),
        ],
    )(logits, t2)
    return idxs.reshape(R)
